# baseline (speedup 1.0000x reference)
import jax
import jax.numpy as jnp
import numpy as np
from jax.experimental import pallas as pl

B, N, C, OUT, K, HID = 4, 2048, 256, 256, 16, 256


def _mlp(x, W1, b1, W2, b2):
    h = jax.nn.relu(jnp.einsum('...i,hi->...h', x, W1) + b1)
    return jnp.einsum('...h,oh->...o', h, W2) + b2


def kernel(x, pos, normal, mask, Wq1, bq1, Wq2, bq2, Wk1, bk1, Wk2, bk2, Wv1, bv1, Wv2, bv2, Wp1, bp1, Wp2, bp2, Wo1, bo1, Wo2, bo2, gamma, beta):
    sq = jnp.sum(pos * pos, axis=-1)
    d2 = sq[:, :, None] + sq[:, None, :] - 2.0 * jnp.einsum('bnd,bmd->bnm', pos, pos)
    dist = jnp.sqrt(jnp.clip(d2, 0.0, None))
    dist = jnp.where(mask == 0, jnp.inf, dist)
    _, idx = jax.lax.top_k(-dist, K)
    q = _mlp(x, Wq1, bq1, Wq2, bq2)
    k = _mlp(x, Wk1, bk1, Wk2, bk2)
    v = _mlp(x, Wv1, bv1, Wv2, bv2)
    residual = q
    gather = jax.vmap(lambda a, i: a[i])
    kg = gather(k, idx)
    vg = gather(v, idx)
    pg = gather(pos, idx)
    pos_rel = pos[:, :, None, :] - pg
    nrm = normal / jnp.clip(jnp.linalg.norm(normal, axis=-1, keepdims=True), 1e-12, None)
    radial = jnp.sqrt(jnp.sum(pos_rel * pos_rel, axis=-1, keepdims=True) + 1e-20)
    dot = jnp.sum(pos_rel * nrm[:, :, None, :], axis=-1, keepdims=True)
    ratio = jnp.clip(dot / (radial + 1e-8), -1.0 + 1e-6, 1.0 - 1e-6)
    theta = jnp.arccos(ratio)
    polar = jnp.concatenate([radial, theta], axis=-1)
    pe = jnp.einsum('...h,oh->...o', jax.nn.relu(jnp.einsum('...i,oi->...o', polar, Wp1) + bp1), Wp2) + bp2
    kg = kg + pe
    vg = vg + pe
    qe = jnp.broadcast_to(q[:, :, None, :], (B, N, K, OUT))
    attn = jax.nn.softmax(jnp.einsum('bnkd,bnjd->bnkj', qe, kg) / np.sqrt(OUT), axis=-1)
    out = jnp.sum(jnp.einsum('bnkj,bnjd->bnkd', attn, vg), axis=2) / K
    out = _mlp(out, Wo1, bo1, Wo2, bo2) + residual
    mu = jnp.mean(out, axis=-1, keepdims=True)
    var = jnp.var(out, axis=-1, keepdims=True)
    out = (out - mu) / jnp.sqrt(var + 1e-6) * gamma + beta
    return out


# speedup vs baseline: 5.0568x; 5.0568x over previous
import functools

import jax
import jax.numpy as jnp
import numpy as np
from jax.experimental import pallas as pl
from jax.experimental.pallas import tpu as pltpu

B, N, C, OUT, K, HID = 4, 2048, 256, 256, 16, 256

_INTERPRET = False

BLK_A = 512
BLK_B = 256
BLK_D = 256


def _qkv_body(x_ref, wq1, bq1, wq2, bq2, wk1, bk1, wk2, bk2,
              wv1, bv1, wv2, bv2, wp2, q_ref, k_ref, v_ref, qp_ref):
    x = x_ref[...]
    hq = jax.nn.relu(jnp.dot(x, wq1[...]) + bq1[...])
    q = jnp.dot(hq, wq2[...]) + bq2[...]
    hk = jax.nn.relu(jnp.dot(x, wk1[...]) + bk1[...])
    k_ref[...] = jnp.dot(hk, wk2[...]) + bk2[...]
    hv = jax.nn.relu(jnp.dot(x, wv1[...]) + bv1[...])
    v_ref[...] = jnp.dot(hv, wv2[...]) + bv2[...]
    q_ref[...] = q
    qp_ref[...] = jnp.dot(q, wp2[...])


def _qkv(xf, Wq1T, bq1, Wq2T, bq2, Wk1T, bk1, Wk2T, bk2, Wv1T, bv1, Wv2T, bv2, Wp2):
    nblk = (B * N) // BLK_A
    row = pl.BlockSpec((BLK_A, C), lambda i: (i, 0))
    full = pl.BlockSpec((C, C), lambda i: (0, 0))
    vec = pl.BlockSpec((C,), lambda i: (0,))
    out_spec = pl.BlockSpec((BLK_A, OUT), lambda i: (i, 0))
    return pl.pallas_call(
        _qkv_body,
        grid=(nblk,),
        in_specs=[row, full, vec, full, vec, full, vec, full, vec,
                  full, vec, full, vec, full],
        out_specs=[out_spec] * 4,
        out_shape=[jax.ShapeDtypeStruct((B * N, OUT), jnp.float32)] * 4,
        interpret=_INTERPRET,
    )(xf, Wq1T, bq1, Wq2T, bq2, Wk1T, bk1, Wk2T, bk2, Wv1T, bv1, Wv2T, bv2, Wp2)


def _topk_body(pos_ref, post_ref, idx_ref):
    pos_blk = pos_ref[0][:, 0:3]
    post = post_ref[0]
    sqm = jnp.sum(post * post, axis=0)
    sqp = jnp.sum(pos_blk * pos_blk, axis=1)
    dotm = jnp.dot(pos_blk, post)
    d2 = sqp[:, None] + sqm[None, :] - 2.0 * dotm
    d2 = jnp.maximum(d2, 0.0)
    iota = jax.lax.broadcasted_iota(jnp.int32, (BLK_B, N), 1)
    cols = []
    cur = d2
    for _ in range(K):
        m = jnp.min(cur, axis=1, keepdims=True)
        cand = jnp.where(cur == m, iota, N)
        pick = jnp.min(cand, axis=1)
        cols.append(pick)
        cur = jnp.where(iota == pick[:, None], jnp.inf, cur)
    idx_ref[0] = jnp.stack(cols, axis=-1)


def _topk(geo):
    nblk = N // BLK_B
    post = jnp.transpose(geo[:, :, 0:3], (0, 2, 1))
    return pl.pallas_call(
        _topk_body,
        grid=(B, nblk),
        in_specs=[pl.BlockSpec((1, BLK_B, 8), lambda b, i: (b, i, 0)),
                  pl.BlockSpec((1, 3, N), lambda b, i: (b, 0, 0))],
        out_specs=pl.BlockSpec((1, BLK_B, K), lambda b, i: (b, i, 0)),
        out_shape=jax.ShapeDtypeStruct((B, N, K), jnp.int32),
        interpret=_INTERPRET,
    )(geo.reshape(B, nblk * BLK_B, 8)[:, :, :],
      post)


_ACOS = (1.5707963050, -0.2145988016, 0.0889789874, -0.0501743046,
         0.0308918810, -0.0170881256, 0.0066700901, -0.0012624911)


def _arccos(x):
    ax = jnp.abs(x)
    p = jnp.full_like(ax, _ACOS[7])
    for c in (_ACOS[6], _ACOS[5], _ACOS[4], _ACOS[3], _ACOS[2], _ACOS[1], _ACOS[0]):
        p = p * ax + c
    r = jnp.sqrt(jnp.maximum(1.0 - ax, 0.0)) * p
    return jnp.where(x < 0.0, np.float32(np.pi) - r, r)


def _attn_body(q_ref, qp_ref, geo_ref, kg_ref, vg_ref, pg_ref,
               wp1_ref, bp1_ref, wp2t_ref, bp2_ref,
               wo1_ref, bo1_ref, wo2_ref, bo2_ref, gamma_ref, beta_ref,
               out_ref):
    q = q_ref[...]
    qp = qp_ref[...]
    geo = geo_ref[...]
    kg = kg_ref[0]
    vg = vg_ref[0]
    pg = pg_ref[0]

    px, py, pz = geo[:, 0], geo[:, 1], geo[:, 2]
    nx, ny, nz = geo[:, 3], geo[:, 4], geo[:, 5]
    nn = jnp.maximum(jnp.sqrt(nx * nx + ny * ny + nz * nz), 1e-12)
    nx, ny, nz = nx / nn, ny / nn, nz / nn

    dx = px[:, None] - pg[:, :, 0]
    dy = py[:, None] - pg[:, :, 1]
    dz = pz[:, None] - pg[:, :, 2]
    radial = jnp.sqrt(dx * dx + dy * dy + dz * dz + 1e-20)
    dot = dx * nx[:, None] + dy * ny[:, None] + dz * nz[:, None]
    ratio = jnp.clip(dot / (radial + 1e-8), -1.0 + 1e-6, 1.0 - 1e-6)
    theta = _arccos(ratio)

    w0 = wp1_ref[0, :]
    w1 = wp1_ref[1, :]
    r3 = jax.nn.relu(radial[:, :, None] * w0[None, None, :]
                     + theta[:, :, None] * w1[None, None, :]
                     + bp1_ref[...][None, None, :])

    logits = (jnp.sum(q[:, None, :] * kg, axis=-1)
              + jnp.sum(qp[:, None, :] * r3, axis=-1)) * (1.0 / 16.0)
    logits = logits - jnp.max(logits, axis=-1, keepdims=True)
    e = jnp.exp(logits)
    a = e / jnp.sum(e, axis=-1, keepdims=True)

    w = jnp.sum(a[:, :, None] * vg, axis=1)
    g = jnp.sum(a[:, :, None] * r3, axis=1)
    attn_out = w + jnp.dot(g, wp2t_ref[...]) + bp2_ref[...][None, :]

    ho = jax.nn.relu(jnp.dot(attn_out, wo1_ref[...]) + bo1_ref[...])
    out = jnp.dot(ho, wo2_ref[...]) + bo2_ref[...] + q
    mu = jnp.mean(out, axis=-1, keepdims=True)
    var = jnp.mean((out - mu) ** 2, axis=-1, keepdims=True)
    out_ref[...] = ((out - mu) / jnp.sqrt(var + 1e-6)
                    * gamma_ref[...][None, :] + beta_ref[...][None, :])


def _attention(q, qp, geo, kg, vg, pg, Wp1, bp1, Wp2T, bp2,
               Wo1T, bo1, Wo2T, bo2, gamma, beta):
    nblk = (B * N) // BLK_D
    row = pl.BlockSpec((BLK_D, OUT), lambda i: (i, 0))
    row8 = pl.BlockSpec((BLK_D, 8), lambda i: (i, 0))
    g3 = pl.BlockSpec((1, BLK_D, K, OUT), lambda i: (i, 0, 0, 0))
    g38 = pl.BlockSpec((1, BLK_D, K, 8), lambda i: (i, 0, 0, 0))
    full = pl.BlockSpec((OUT, OUT), lambda i: (0, 0))
    vec = pl.BlockSpec((OUT,), lambda i: (0,))
    return pl.pallas_call(
        _attn_body,
        grid=(nblk,),
        in_specs=[row, row, row8, g3, g3, g38,
                  pl.BlockSpec((2, OUT), lambda i: (0, 0)), vec, full, vec,
                  full, vec, full, vec, vec, vec],
        out_specs=row,
        out_shape=jax.ShapeDtypeStruct((B * N, OUT), jnp.float32),
        interpret=_INTERPRET,
    )(q, qp, geo,
      kg.reshape(nblk, BLK_D, K, OUT), vg.reshape(nblk, BLK_D, K, OUT),
      pg.reshape(nblk, BLK_D, K, 8),
      Wp1, bp1, Wp2T, bp2, Wo1T, bo1, Wo2T, bo2, gamma, beta)


def kernel(x, pos, normal, mask, Wq1, bq1, Wq2, bq2, Wk1, bk1, Wk2, bk2,
           Wv1, bv1, Wv2, bv2, Wp1, bp1, Wp2, bp2, Wo1, bo1, Wo2, bo2,
           gamma, beta):
    xf = x.reshape(B * N, C)
    q, k, v, qp = _qkv(xf, Wq1.T, bq1, Wq2.T, bq2, Wk1.T, bk1, Wk2.T, bk2,
                       Wv1.T, bv1, Wv2.T, bv2, Wp2)

    geo = jnp.concatenate(
        [pos, normal, jnp.zeros((B, N, 2), jnp.float32)], axis=-1)
    idx = _topk(geo)

    flat_idx = (idx + (jnp.arange(B, dtype=jnp.int32) * N)[:, None, None]
                ).reshape(B * N * K)
    kg = k[flat_idx]
    vg = v[flat_idx]
    pg = geo.reshape(B * N, 8)[flat_idx]

    out = _attention(q, qp, geo.reshape(B * N, 8), kg, vg, pg,
                     Wp1.T, bp1, Wp2.T, bp2, Wo1.T, bo1, Wo2.T, bo2,
                     gamma, beta)
    return out.reshape(B, N, OUT)


# speedup vs baseline: 12.6729x; 2.5061x over previous
import functools

import jax
import jax.numpy as jnp
import numpy as np
from jax import lax
from jax.experimental import pallas as pl
from jax.experimental.pallas import tpu as pltpu
from jax.experimental.pallas import tpu_sc as plsc

B, N, C, OUT, K, HID = 4, 2048, 256, 256, 16, 256

_INTERPRET = False

BLK_A = 512
BLK_B = 256
BLK_D = 256


def _qkv_body(x_ref, wq1, bq1, wq2, bq2, wk1, bk1, wk2, bk2,
              wv1, bv1, wv2, bv2, wp2, q_ref, k_ref, v_ref, qp_ref):
    x = x_ref[...]
    hq = jax.nn.relu(jnp.dot(x, wq1[...]) + bq1[...])
    q = jnp.dot(hq, wq2[...]) + bq2[...]
    hk = jax.nn.relu(jnp.dot(x, wk1[...]) + bk1[...])
    k_ref[...] = jnp.dot(hk, wk2[...]) + bk2[...]
    hv = jax.nn.relu(jnp.dot(x, wv1[...]) + bv1[...])
    v_ref[...] = jnp.dot(hv, wv2[...]) + bv2[...]
    q_ref[...] = q
    qp_ref[...] = jnp.dot(q, wp2[...])


def _qkv(xf, Wq1T, bq1, Wq2T, bq2, Wk1T, bk1, Wk2T, bk2, Wv1T, bv1, Wv2T, bv2, Wp2):
    nblk = (B * N) // BLK_A
    row = pl.BlockSpec((BLK_A, C), lambda i: (i, 0))
    full = pl.BlockSpec((C, C), lambda i: (0, 0))
    vec = pl.BlockSpec((C,), lambda i: (0,))
    out_spec = pl.BlockSpec((BLK_A, OUT), lambda i: (i, 0))
    return pl.pallas_call(
        _qkv_body,
        grid=(nblk,),
        in_specs=[row, full, vec, full, vec, full, vec, full, vec,
                  full, vec, full, vec, full],
        out_specs=[out_spec] * 4,
        out_shape=[jax.ShapeDtypeStruct((B * N, OUT), jnp.float32)] * 4,
        interpret=_INTERPRET,
    )(xf, Wq1T, bq1, Wq2T, bq2, Wk1T, bk1, Wk2T, bk2, Wv1T, bv1, Wv2T, bv2, Wp2)


def _topk_body(pos_ref, post_ref, idx_ref):
    pos_blk = pos_ref[0][:, 0:3]
    post = post_ref[0]
    sqm = jnp.sum(post * post, axis=0)
    sqp = jnp.sum(pos_blk * pos_blk, axis=1)
    dotm = jnp.dot(pos_blk, post)
    d2 = sqp[:, None] + sqm[None, :] - 2.0 * dotm
    d2 = jnp.maximum(d2, 0.0)
    iota = jax.lax.broadcasted_iota(jnp.int32, (BLK_B, N), 1)
    cols = []
    cur = d2
    for _ in range(K):
        m = jnp.min(cur, axis=1, keepdims=True)
        cand = jnp.where(cur == m, iota, N)
        pick = jnp.min(cand, axis=1)
        cols.append(pick)
        cur = jnp.where(iota == pick[:, None], jnp.inf, cur)
    idx_ref[0] = jnp.stack(cols, axis=-1) + pl.program_id(0) * N


def _topk(geo):
    nblk = N // BLK_B
    post = jnp.transpose(geo[:, :, 0:3], (0, 2, 1))
    return pl.pallas_call(
        _topk_body,
        grid=(B, nblk),
        in_specs=[pl.BlockSpec((1, BLK_B, GEO_D), lambda b, i: (b, i, 0)),
                  pl.BlockSpec((1, 3, N), lambda b, i: (b, 0, 0))],
        out_specs=pl.BlockSpec((1, BLK_B, K), lambda b, i: (b, i, 0)),
        out_shape=jax.ShapeDtypeStruct((B, N, K), jnp.int32),
        interpret=_INTERPRET,
    )(geo, post)


GEO_D = 16
SC_CH = 128


def _sc_gather(kf, vf, geof, flat_idx):
    total = flat_idx.shape[0]
    info = plsc.get_sparse_core_info()
    nw = info.num_cores * info.num_subcores
    L = info.num_lanes
    per_w = total // nw
    n_ch = per_w // SC_CH
    mesh = plsc.VectorSubcoreMesh(core_axis_name="c", subcore_axis_name="s")

    @functools.partial(
        pl.kernel, mesh=mesh,
        compiler_params=pltpu.CompilerParams(needs_layout_passes=False),
        out_type=[jax.ShapeDtypeStruct((total, OUT), jnp.float32),
                  jax.ShapeDtypeStruct((total, OUT), jnp.float32),
                  jax.ShapeDtypeStruct((total,), jnp.float32),
                  jax.ShapeDtypeStruct((total,), jnp.float32),
                  jax.ShapeDtypeStruct((total,), jnp.float32)],
        scratch_types=[pltpu.VMEM((SC_CH,), jnp.int32),
                       pltpu.VMEM((SC_CH, OUT), jnp.float32),
                       pltpu.VMEM((SC_CH, OUT), jnp.float32),
                       pltpu.VMEM((N * GEO_D,), jnp.float32),
                       pltpu.VMEM((SC_CH,), jnp.float32),
                       pltpu.VMEM((SC_CH,), jnp.float32),
                       pltpu.VMEM((SC_CH,), jnp.float32),
                       pltpu.SemaphoreType.DMA,
                       pltpu.SemaphoreType.DMA],
    )
    def gk(kf_h, vf_h, geo_h, idx_h, kg_h, vg_h, px_h, py_h, pz_h,
           idx_v, kr, vr, slab, pxr, pyr, pzr, s1, s2):
        wid = lax.axis_index("s") * info.num_cores + lax.axis_index("c")
        base = wid * per_w
        batch = base // (N * K)
        row0 = batch * N
        pltpu.sync_copy(geo_h.at[pl.ds(row0 * GEO_D, N * GEO_D)], slab)

        def body(i, carry):
            off = base + i * SC_CH
            pltpu.sync_copy(idx_h.at[pl.ds(off, SC_CH)], idx_v)
            c1 = pltpu.async_copy(kf_h.at[idx_v], kr, s1)
            c2 = pltpu.async_copy(vf_h.at[idx_v], vr, s2)
            for j in range(SC_CH // L):
                addr = (idx_v[pl.ds(j * L, L)] - row0) * GEO_D
                pxr[pl.ds(j * L, L)] = plsc.load_gather(slab, [addr])
                pyr[pl.ds(j * L, L)] = plsc.load_gather(slab, [addr + 1])
                pzr[pl.ds(j * L, L)] = plsc.load_gather(slab, [addr + 2])
            c1.wait()
            c2.wait()
            pltpu.sync_copy(kr, kg_h.at[pl.ds(off, SC_CH)])
            pltpu.sync_copy(vr, vg_h.at[pl.ds(off, SC_CH)])
            pltpu.sync_copy(pxr, px_h.at[pl.ds(off, SC_CH)])
            pltpu.sync_copy(pyr, py_h.at[pl.ds(off, SC_CH)])
            pltpu.sync_copy(pzr, pz_h.at[pl.ds(off, SC_CH)])
            return carry

        lax.fori_loop(0, n_ch, body, 0)

    return gk(kf, vf, geof, flat_idx)


_ACOS = (1.5707963050, -0.2145988016, 0.0889789874, -0.0501743046,
         0.0308918810, -0.0170881256, 0.0066700901, -0.0012624911)


def _arccos(x):
    ax = jnp.abs(x)
    p = jnp.full_like(ax, _ACOS[7])
    for c in (_ACOS[6], _ACOS[5], _ACOS[4], _ACOS[3], _ACOS[2], _ACOS[1], _ACOS[0]):
        p = p * ax + c
    r = jnp.sqrt(jnp.maximum(1.0 - ax, 0.0)) * p
    return jnp.where(x < 0.0, np.float32(np.pi) - r, r)


def _attn_body(q_ref, qp_ref, geo_ref, kg_ref, vg_ref,
               pgx_ref, pgy_ref, pgz_ref,
               wp1_ref, bp1_ref, wp2t_ref, bp2_ref,
               wo1_ref, bo1_ref, wo2_ref, bo2_ref, gamma_ref, beta_ref,
               out_ref):
    q = q_ref[...]
    qp = qp_ref[...]
    geo = geo_ref[...]
    kg = kg_ref[0]
    vg = vg_ref[0]

    px, py, pz = geo[:, 0], geo[:, 1], geo[:, 2]
    nx, ny, nz = geo[:, 3], geo[:, 4], geo[:, 5]
    nn = jnp.maximum(jnp.sqrt(nx * nx + ny * ny + nz * nz), 1e-12)
    nx, ny, nz = nx / nn, ny / nn, nz / nn

    dx = px[:, None] - pgx_ref[0]
    dy = py[:, None] - pgy_ref[0]
    dz = pz[:, None] - pgz_ref[0]
    radial = jnp.sqrt(dx * dx + dy * dy + dz * dz + 1e-20)
    dot = dx * nx[:, None] + dy * ny[:, None] + dz * nz[:, None]
    ratio = jnp.clip(dot / (radial + 1e-8), -1.0 + 1e-6, 1.0 - 1e-6)
    theta = _arccos(ratio)

    w0 = wp1_ref[0, :]
    w1 = wp1_ref[1, :]
    r3 = jax.nn.relu(radial[:, :, None] * w0[None, None, :]
                     + theta[:, :, None] * w1[None, None, :]
                     + bp1_ref[...][None, None, :])

    logits = (jnp.sum(q[:, None, :] * kg, axis=-1)
              + jnp.sum(qp[:, None, :] * r3, axis=-1)) * (1.0 / 16.0)
    logits = logits - jnp.max(logits, axis=-1, keepdims=True)
    e = jnp.exp(logits)
    a = e / jnp.sum(e, axis=-1, keepdims=True)

    w = jnp.sum(a[:, :, None] * vg, axis=1)
    g = jnp.sum(a[:, :, None] * r3, axis=1)
    attn_out = w + jnp.dot(g, wp2t_ref[...]) + bp2_ref[...][None, :]

    ho = jax.nn.relu(jnp.dot(attn_out, wo1_ref[...]) + bo1_ref[...])
    out = jnp.dot(ho, wo2_ref[...]) + bo2_ref[...] + q
    mu = jnp.mean(out, axis=-1, keepdims=True)
    var = jnp.mean((out - mu) ** 2, axis=-1, keepdims=True)
    out_ref[...] = ((out - mu) / jnp.sqrt(var + 1e-6)
                    * gamma_ref[...][None, :] + beta_ref[...][None, :])


def _attention(q, qp, geo, kg, vg, pgx, pgy, pgz, Wp1, bp1, Wp2T, bp2,
               Wo1T, bo1, Wo2T, bo2, gamma, beta):
    nblk = (B * N) // BLK_D
    row = pl.BlockSpec((BLK_D, OUT), lambda i: (i, 0))
    row8 = pl.BlockSpec((BLK_D, GEO_D), lambda i: (i, 0))
    g3 = pl.BlockSpec((1, BLK_D, K, OUT), lambda i: (i, 0, 0, 0))
    gk = pl.BlockSpec((1, BLK_D, K), lambda i: (i, 0, 0))
    full = pl.BlockSpec((OUT, OUT), lambda i: (0, 0))
    vec = pl.BlockSpec((OUT,), lambda i: (0,))
    return pl.pallas_call(
        _attn_body,
        grid=(nblk,),
        in_specs=[row, row, row8, g3, g3, gk, gk, gk,
                  pl.BlockSpec((2, OUT), lambda i: (0, 0)), vec, full, vec,
                  full, vec, full, vec, vec, vec],
        out_specs=row,
        out_shape=jax.ShapeDtypeStruct((B * N, OUT), jnp.float32),
        interpret=_INTERPRET,
    )(q, qp, geo,
      kg.reshape(nblk, BLK_D, K, OUT), vg.reshape(nblk, BLK_D, K, OUT),
      pgx.reshape(nblk, BLK_D, K), pgy.reshape(nblk, BLK_D, K),
      pgz.reshape(nblk, BLK_D, K),
      Wp1, bp1, Wp2T, bp2, Wo1T, bo1, Wo2T, bo2, gamma, beta)


def kernel(x, pos, normal, mask, Wq1, bq1, Wq2, bq2, Wk1, bk1, Wk2, bk2,
           Wv1, bv1, Wv2, bv2, Wp1, bp1, Wp2, bp2, Wo1, bo1, Wo2, bo2,
           gamma, beta):
    xf = x.reshape(B * N, C)
    q, k, v, qp = _qkv(xf, Wq1.T, bq1, Wq2.T, bq2, Wk1.T, bk1, Wk2.T, bk2,
                       Wv1.T, bv1, Wv2.T, bv2, Wp2)

    geo = jnp.concatenate(
        [pos, normal, jnp.zeros((B, N, GEO_D - 6), jnp.float32)],
        axis=-1)
    idx = _topk(geo)

    flat_idx = idx.reshape(B * N * K)
    kg, vg, pgx, pgy, pgz = _sc_gather(k, v, geo.reshape(B * N * GEO_D),
                                       flat_idx)

    out = _attention(q, qp, geo.reshape(B * N, GEO_D), kg, vg, pgx, pgy, pgz,
                     Wp1.T, bp1, Wp2.T, bp2, Wo1.T, bo1, Wo2.T, bo2,
                     gamma, beta)
    return out.reshape(B, N, OUT)
